# baseline (speedup 1.0000x reference)
import functools

import jax
import jax.numpy as jnp
from jax import lax
from jax.experimental import pallas as pl
from jax.experimental.pallas import tpu as pltpu
from jax.experimental.pallas import tpu_sc as plsc

ALPHA = 0.5

NC = 2
NS = 16
CH = 128
N_PAD = 10240
RPT = N_PAD // NS
ZR = 32
KB = 4

_MESH = plsc.VectorSubcoreMesh(
    core_axis_name="c", subcore_axis_name="s", num_cores=NC, num_subcores=NS)


def _zero_fill_1d(ref, n):
  def body(i, _):
    ref[pl.ds(i * 16, 16)] = jnp.zeros((16,), jnp.float32)
    return 0
  lax.fori_loop(0, n // 16, body, 0)


def _zero_fill_2d(ref, rows, cols):
  def body(i, _):
    def inner(j, _):
      ref[i, pl.ds(j * 16, 16)] = jnp.zeros((16,), jnp.float32)
      return 0
    lax.fori_loop(0, cols // 16, inner, 0)
    return 0
  lax.fori_loop(0, rows, body, 0)


def _deg_body(kch, idx_hbm, deg_hbm, idx_v, ones_v, zb_v, deg_sh):
  cid = lax.axis_index("c")
  sid = lax.axis_index("s")
  def ones_body(i, _):
    ones_v[pl.ds(i * 16, 16)] = jnp.ones((16,), jnp.float32)
    return 0
  lax.fori_loop(0, CH // 16, ones_body, 0)
  _zero_fill_1d(zb_v, RPT)
  pltpu.sync_copy(zb_v, deg_sh.at[pl.ds(sid * RPT, RPT)])
  plsc.subcore_barrier()
  pltpu.sync_copy(idx_hbm.at[cid, sid], idx_v)
  def body(j, _):
    pltpu.sync_copy(ones_v, deg_sh.at[idx_v.at[j]], add=True)
    return 0
  lax.fori_loop(0, kch, body, 0)
  plsc.subcore_barrier()
  pltpu.sync_copy(deg_sh.at[pl.ds(sid * RPT, RPT)],
                  deg_hbm.at[cid, pl.ds(sid * RPT, RPT)])


def _agg_body(kch, d, sidx_hbm, didx_hbm, y_hbm, acc_hbm,
              sidx_v, didx_v, buf0_v, buf1_v, zb_v, acc_sh, sem0, sem1):
  bufs = (buf0_v, buf1_v)
  sems = (sem0, sem1)
  cid = lax.axis_index("c")
  sid = lax.axis_index("s")
  _zero_fill_2d(zb_v, ZR, d)
  def zbody(t, _):
    pltpu.sync_copy(zb_v, acc_sh.at[pl.ds(sid * RPT + t * ZR, ZR)])
    return 0
  lax.fori_loop(0, RPT // ZR, zbody, 0)
  plsc.subcore_barrier()
  qch = kch // 4
  for h in range(4):
    pltpu.sync_copy(sidx_hbm.at[cid, sid, h], sidx_v)
    pltpu.sync_copy(didx_hbm.at[cid, sid, h], didx_v)
    def outer(g, _):
      j0 = g * 2
      d0 = pltpu.async_copy(y_hbm.at[sidx_v.at[j0]], bufs[0], sems[0])
      d1 = pltpu.async_copy(y_hbm.at[sidx_v.at[j0 + 1]], bufs[1], sems[1])
      d0.wait()
      pltpu.sync_copy(bufs[0], acc_sh.at[didx_v.at[j0]], add=True)
      d1.wait()
      pltpu.sync_copy(bufs[1], acc_sh.at[didx_v.at[j0 + 1]], add=True)
      return 0
    lax.fori_loop(0, qch // 2, outer, 0)
  plsc.subcore_barrier()
  pltpu.sync_copy(acc_sh.at[pl.ds(sid * RPT, RPT)],
                  acc_hbm.at[pl.ds(cid * N_PAD + sid * RPT, RPT)])


def _prep_body(x_ref, wsd_ref, wds_ref, od_ref, id_ref,
               y_ref, oinv_ref, iinv_ref):
  x = x_ref[...]
  od = od_ref[...]
  idg = id_ref[...]
  oinv = jnp.where(od > 0, lax.rsqrt(od), 0.0)
  iinv = jnp.where(idg > 0, lax.rsqrt(idg), 0.0)
  dn = (((1,), (1,)), ((), ()))
  y_ref[0] = (1.0 - ALPHA) * iinv * lax.dot_general(
      x, wsd_ref[...], dn, precision=lax.Precision.HIGHEST,
      preferred_element_type=jnp.float32)
  y_ref[1] = ALPHA * oinv * lax.dot_general(
      x, wds_ref[...], dn, precision=lax.Precision.HIGHEST,
      preferred_element_type=jnp.float32)
  oinv_ref[...] = oinv
  iinv_ref[...] = iinv


def _final_body(acc_ref, oinv_ref, iinv_ref, bsd_ref, bds_ref, out_ref):
  b = (1.0 - ALPHA) * bsd_ref[...] + ALPHA * bds_ref[...]
  out_ref[...] = (oinv_ref[...] * acc_ref[0]
                  + iinv_ref[...] * acc_ref[1]
                  + b[None, :])


def kernel(x, edge_index, W_sd, b_sd, W_ds, b_ds):
  n, d = x.shape
  e = edge_index.shape[1]
  kch = 8 * ((e + NS * CH * 8 - 1) // (NS * CH * 8))
  epw = CH * kch
  e_pad = NS * epw

  row = edge_index[0].astype(jnp.int32)
  col = edge_index[1].astype(jnp.int32)
  pad = jnp.full((e_pad - e,), N_PAD - 1, jnp.int32)
  row_p = jnp.concatenate([row, pad])
  col_p = jnp.concatenate([col, pad])
  dst_idx = jnp.stack([row_p, col_p]).reshape(NC, NS, kch, CH)
  src_idx = jnp.stack([col_p, row_p + N_PAD]).reshape(NC, NS, kch, CH)

  x_pad = jnp.pad(x, ((0, N_PAD - n), (0, 0)))

  deg_call = pl.kernel(
      functools.partial(_deg_body, kch),
      out_type=jax.ShapeDtypeStruct((NC, N_PAD), jnp.float32),
      mesh=_MESH,
      scratch_types=[
          pltpu.VMEM((kch, CH), jnp.int32),
          pltpu.VMEM((CH,), jnp.float32),
          pltpu.VMEM((RPT,), jnp.float32),
          pltpu.VMEM_SHARED((N_PAD,), jnp.float32),
      ],
  )
  deg = deg_call(dst_idx)

  bn = 1024
  grid = (N_PAD // bn,)
  deg3 = deg.reshape(NC, N_PAD, 1)
  y3, oinv, iinv = pl.pallas_call(
      _prep_body,
      grid=grid,
      in_specs=[
          pl.BlockSpec((bn, d), lambda i: (i, 0)),
          pl.BlockSpec((d, d), lambda i: (0, 0)),
          pl.BlockSpec((d, d), lambda i: (0, 0)),
          pl.BlockSpec((bn, 1), lambda i: (i, 0)),
          pl.BlockSpec((bn, 1), lambda i: (i, 0)),
      ],
      out_specs=[
          pl.BlockSpec((NC, bn, d), lambda i: (0, i, 0)),
          pl.BlockSpec((bn, 1), lambda i: (i, 0)),
          pl.BlockSpec((bn, 1), lambda i: (i, 0)),
      ],
      out_shape=[
          jax.ShapeDtypeStruct((NC, N_PAD, d), jnp.float32),
          jax.ShapeDtypeStruct((N_PAD, 1), jnp.float32),
          jax.ShapeDtypeStruct((N_PAD, 1), jnp.float32),
      ],
  )(x_pad, W_sd, W_ds, deg3[0], deg3[1])
  y_cat = y3.reshape(NC * N_PAD, d)

  agg_call = pl.kernel(
      functools.partial(_agg_body, kch, d),
      out_type=jax.ShapeDtypeStruct((NC * N_PAD, d), jnp.float32),
      mesh=_MESH,
      compiler_params=pltpu.CompilerParams(use_tc_tiling_on_sc=False),
      scratch_types=[
          pltpu.VMEM((kch // 4, CH), jnp.int32),
          pltpu.VMEM((kch // 4, CH), jnp.int32),
          pltpu.VMEM((CH, d), jnp.float32),
          pltpu.VMEM((CH, d), jnp.float32),
          pltpu.VMEM((ZR, d), jnp.float32),
          pltpu.VMEM_SHARED((N_PAD, d), jnp.float32),
          pltpu.SemaphoreType.DMA,
          pltpu.SemaphoreType.DMA,
      ],
  )
  acc = agg_call(src_idx.reshape(NC, NS, 4, kch // 4, CH),
                 dst_idx.reshape(NC, NS, 4, kch // 4, CH),
                 y_cat).reshape(NC, N_PAD, d)

  out_pad = pl.pallas_call(
      _final_body,
      grid=grid,
      in_specs=[
          pl.BlockSpec((NC, bn, d), lambda i: (0, i, 0)),
          pl.BlockSpec((bn, 1), lambda i: (i, 0)),
          pl.BlockSpec((bn, 1), lambda i: (i, 0)),
          pl.BlockSpec((d,), lambda i: (0,)),
          pl.BlockSpec((d,), lambda i: (0,)),
      ],
      out_specs=pl.BlockSpec((bn, d), lambda i: (i, 0)),
      out_shape=jax.ShapeDtypeStruct((N_PAD, d), jnp.float32),
  )(acc, oinv, iinv, b_sd, b_ds)

  return out_pad[:n]


# speedup vs baseline: 1.3299x; 1.3299x over previous
import functools

import jax
import jax.numpy as jnp
from jax import lax
from jax.experimental import pallas as pl
from jax.experimental.pallas import tpu as pltpu
from jax.experimental.pallas import tpu_sc as plsc

ALPHA = 0.5

NC = 2
NS = 16
CH = 128
N_PAD = 10240
RPT = N_PAD // NS
ZR = 32
KB = 4

_MESH = plsc.VectorSubcoreMesh(
    core_axis_name="c", subcore_axis_name="s", num_cores=NC, num_subcores=NS)


def _zero_fill_1d(ref, n):
  def body(i, _):
    ref[pl.ds(i * 16, 16)] = jnp.zeros((16,), jnp.float32)
    return 0
  lax.fori_loop(0, n // 16, body, 0)


def _zero_fill_2d(ref, rows, cols):
  def body(i, _):
    def inner(j, _):
      ref[i, pl.ds(j * 16, 16)] = jnp.zeros((16,), jnp.float32)
      return 0
    lax.fori_loop(0, cols // 16, inner, 0)
    return 0
  lax.fori_loop(0, rows, body, 0)


def _deg_body(kch, idx_hbm, deg_hbm, idx_v, ones_v, zb_v, deg_sh):
  cid = lax.axis_index("c")
  sid = lax.axis_index("s")
  def ones_body(i, _):
    ones_v[pl.ds(i * 16, 16)] = jnp.ones((16,), jnp.float32)
    return 0
  lax.fori_loop(0, CH // 16, ones_body, 0)
  _zero_fill_1d(zb_v, RPT)
  pltpu.sync_copy(zb_v, deg_sh.at[pl.ds(sid * RPT, RPT)])
  plsc.subcore_barrier()
  pltpu.sync_copy(idx_hbm.at[cid, sid], idx_v)
  def body(j, _):
    pltpu.sync_copy(ones_v, deg_sh.at[idx_v.at[j]], add=True)
    return 0
  lax.fori_loop(0, kch, body, 0)
  plsc.subcore_barrier()
  pltpu.sync_copy(deg_sh.at[pl.ds(sid * RPT, RPT)],
                  deg_hbm.at[cid, pl.ds(sid * RPT, RPT)])


def _agg_body(kch, d, sidx_hbm, didx_hbm, y_hbm, acc_hbm,
              sidx_v, didx_v, buf_v, zb_v, acc_sh, sem):
  cid = lax.axis_index("c")
  sid = lax.axis_index("s")
  _zero_fill_2d(zb_v, ZR, d)
  def zbody(t, _):
    pltpu.sync_copy(zb_v, acc_sh.at[pl.ds(sid * RPT + t * ZR, ZR)])
    return 0
  lax.fori_loop(0, RPT // ZR, zbody, 0)
  plsc.subcore_barrier()
  hch = kch // 2
  for h in range(2):
    pltpu.sync_copy(sidx_hbm.at[cid, sid, h], sidx_v)
    pltpu.sync_copy(didx_hbm.at[cid, sid, h], didx_v)
    def body(j, _):
      pltpu.async_copy(y_hbm.at[sidx_v.at[j]], buf_v, sem).wait()
      pltpu.sync_copy(buf_v, acc_sh.at[didx_v.at[j]], add=True)
      return 0
    lax.fori_loop(0, hch, body, 0)
  plsc.subcore_barrier()
  pltpu.sync_copy(acc_sh.at[pl.ds(sid * RPT, RPT)],
                  acc_hbm.at[pl.ds(cid * N_PAD + sid * RPT, RPT)])


def _prep_body(x_ref, wsd_ref, wds_ref, od_ref, id_ref,
               y_ref, oinv_ref, iinv_ref):
  x = x_ref[...]
  od = od_ref[...]
  idg = id_ref[...]
  oinv = jnp.where(od > 0, lax.rsqrt(od), 0.0)
  iinv = jnp.where(idg > 0, lax.rsqrt(idg), 0.0)
  dn = (((1,), (1,)), ((), ()))
  y_ref[0] = (1.0 - ALPHA) * iinv * lax.dot_general(
      x, wsd_ref[...], dn, precision=lax.Precision.HIGHEST,
      preferred_element_type=jnp.float32)
  y_ref[1] = ALPHA * oinv * lax.dot_general(
      x, wds_ref[...], dn, precision=lax.Precision.HIGHEST,
      preferred_element_type=jnp.float32)
  oinv_ref[...] = oinv
  iinv_ref[...] = iinv


def _final_body(acc_ref, oinv_ref, iinv_ref, bsd_ref, bds_ref, out_ref):
  b = (1.0 - ALPHA) * bsd_ref[...] + ALPHA * bds_ref[...]
  out_ref[...] = (oinv_ref[...] * acc_ref[0]
                  + iinv_ref[...] * acc_ref[1]
                  + b[None, :])


def kernel(x, edge_index, W_sd, b_sd, W_ds, b_ds):
  n, d = x.shape
  e = edge_index.shape[1]
  kch = 2 * ((e + NS * CH * 2 - 1) // (NS * CH * 2))
  epw = CH * kch
  e_pad = NS * epw

  row = edge_index[0].astype(jnp.int32)
  col = edge_index[1].astype(jnp.int32)
  pad = jnp.full((e_pad - e,), N_PAD - 1, jnp.int32)
  row_p = jnp.concatenate([row, pad])
  col_p = jnp.concatenate([col, pad])
  dst_idx = jnp.stack([row_p, col_p]).reshape(NC, NS, kch, CH)
  src_idx = jnp.stack([col_p, row_p + N_PAD]).reshape(NC, NS, kch, CH)

  deg_call = pl.kernel(
      functools.partial(_deg_body, kch),
      out_type=jax.ShapeDtypeStruct((NC, N_PAD), jnp.float32),
      mesh=_MESH,
      scratch_types=[
          pltpu.VMEM((kch, CH), jnp.int32),
          pltpu.VMEM((CH,), jnp.float32),
          pltpu.VMEM((RPT,), jnp.float32),
          pltpu.VMEM_SHARED((N_PAD,), jnp.float32),
      ],
  )
  deg = deg_call(dst_idx)

  bn = 1000
  grid = (n // bn,)
  deg3 = deg.reshape(NC, N_PAD, 1)
  y3, oinv, iinv = pl.pallas_call(
      _prep_body,
      grid=grid,
      in_specs=[
          pl.BlockSpec((bn, d), lambda i: (i, 0)),
          pl.BlockSpec((d, d), lambda i: (0, 0)),
          pl.BlockSpec((d, d), lambda i: (0, 0)),
          pl.BlockSpec((bn, 1), lambda i: (i, 0)),
          pl.BlockSpec((bn, 1), lambda i: (i, 0)),
      ],
      out_specs=[
          pl.BlockSpec((NC, bn, d), lambda i: (0, i, 0)),
          pl.BlockSpec((bn, 1), lambda i: (i, 0)),
          pl.BlockSpec((bn, 1), lambda i: (i, 0)),
      ],
      out_shape=[
          jax.ShapeDtypeStruct((NC, N_PAD, d), jnp.float32),
          jax.ShapeDtypeStruct((N_PAD, 1), jnp.float32),
          jax.ShapeDtypeStruct((N_PAD, 1), jnp.float32),
      ],
  )(x, W_sd, W_ds, deg3[0], deg3[1])
  y_cat = y3.reshape(NC * N_PAD, d)

  agg_call = pl.kernel(
      functools.partial(_agg_body, kch, d),
      out_type=jax.ShapeDtypeStruct((NC * N_PAD, d), jnp.float32),
      mesh=_MESH,
      compiler_params=pltpu.CompilerParams(use_tc_tiling_on_sc=False),
      scratch_types=[
          pltpu.VMEM((kch // 2, CH), jnp.int32),
          pltpu.VMEM((kch // 2, CH), jnp.int32),
          pltpu.VMEM((CH, d), jnp.float32),
          pltpu.VMEM((ZR, d), jnp.float32),
          pltpu.VMEM_SHARED((N_PAD, d), jnp.float32),
          pltpu.SemaphoreType.DMA,
      ],
  )
  acc = agg_call(src_idx.reshape(NC, NS, 2, kch // 2, CH),
                 dst_idx.reshape(NC, NS, 2, kch // 2, CH),
                 y_cat).reshape(NC, N_PAD, d)

  return pl.pallas_call(
      _final_body,
      grid=grid,
      in_specs=[
          pl.BlockSpec((NC, bn, d), lambda i: (0, i, 0)),
          pl.BlockSpec((bn, 1), lambda i: (i, 0)),
          pl.BlockSpec((bn, 1), lambda i: (i, 0)),
          pl.BlockSpec((d,), lambda i: (0,)),
          pl.BlockSpec((d,), lambda i: (0,)),
      ],
      out_specs=pl.BlockSpec((bn, d), lambda i: (i, 0)),
      out_shape=jax.ShapeDtypeStruct((n, d), jnp.float32),
  )(acc, oinv, iinv, b_sd, b_ds)


# speedup vs baseline: 1.3747x; 1.0337x over previous
import functools

import jax
import jax.numpy as jnp
from jax import lax
from jax.experimental import pallas as pl
from jax.experimental.pallas import tpu as pltpu
from jax.experimental.pallas import tpu_sc as plsc

ALPHA = 0.5

NC = 2
NS = 16
CH = 128
N_PAD = 10240
RPT = N_PAD // NS
ZR = 32
KB = 4

_MESH = plsc.VectorSubcoreMesh(
    core_axis_name="c", subcore_axis_name="s", num_cores=NC, num_subcores=NS)


def _zero_fill_1d(ref, n):
  def body(i, _):
    ref[pl.ds(i * 16, 16)] = jnp.zeros((16,), jnp.float32)
    return 0
  lax.fori_loop(0, n // 16, body, 0)


def _zero_fill_2d(ref, rows, cols):
  def body(i, _):
    def inner(j, _):
      ref[i, pl.ds(j * 16, 16)] = jnp.zeros((16,), jnp.float32)
      return 0
    lax.fori_loop(0, cols // 16, inner, 0)
    return 0
  lax.fori_loop(0, rows, body, 0)


def _deg_body(kch, idx_hbm, deg_hbm, idx_v, ones_v, zb_v, deg_sh):
  cid = lax.axis_index("c")
  sid = lax.axis_index("s")
  def ones_body(i, _):
    ones_v[pl.ds(i * 16, 16)] = jnp.ones((16,), jnp.float32)
    return 0
  lax.fori_loop(0, CH // 16, ones_body, 0)
  _zero_fill_1d(zb_v, RPT)
  pltpu.sync_copy(zb_v, deg_sh.at[pl.ds(sid * RPT, RPT)])
  plsc.subcore_barrier()
  pltpu.sync_copy(idx_hbm.at[cid, sid], idx_v)
  def body(j, _):
    pltpu.sync_copy(ones_v, deg_sh.at[idx_v.at[j]], add=True)
    return 0
  lax.fori_loop(0, kch, body, 0)
  plsc.subcore_barrier()
  pltpu.sync_copy(deg_sh.at[pl.ds(sid * RPT, RPT)],
                  deg_hbm.at[cid, pl.ds(sid * RPT, RPT)])


def _agg_body(kch, d, sidx_hbm, didx_hbm, y_hbm, acc_hbm,
              sidx_v, didx_v, buf_v, zb_v, acc_sh, sem):
  cid = lax.axis_index("c")
  sid = lax.axis_index("s")
  _zero_fill_2d(zb_v, ZR, d)
  def zbody(t, _):
    pltpu.sync_copy(zb_v, acc_sh.at[pl.ds(sid * RPT + t * ZR, ZR)])
    return 0
  lax.fori_loop(0, RPT // ZR, zbody, 0)
  plsc.subcore_barrier()
  hch = kch // 2
  for h in range(2):
    pltpu.sync_copy(sidx_hbm.at[cid, sid, h], sidx_v)
    pltpu.sync_copy(didx_hbm.at[cid, sid, h], didx_v)
    def body(j, _):
      pltpu.async_copy(y_hbm.at[sidx_v.at[j]], buf_v, sem).wait()
      pltpu.sync_copy(buf_v, acc_sh.at[didx_v.at[j]], add=True)
      return 0
    lax.fori_loop(0, hch, body, 0)
  plsc.subcore_barrier()
  pltpu.sync_copy(acc_sh.at[pl.ds(sid * RPT, RPT)],
                  acc_hbm.at[pl.ds(cid * N_PAD + sid * RPT, RPT)])


def _prep_body(x_ref, wsd_ref, wds_ref, od_ref, id_ref,
               y_ref, oinv_ref, iinv_ref):
  x = x_ref[...]
  od = od_ref[...]
  idg = id_ref[...]
  oinv = jnp.where(od > 0, lax.rsqrt(od), 0.0)
  iinv = jnp.where(idg > 0, lax.rsqrt(idg), 0.0)
  dn = (((1,), (1,)), ((), ()))
  y_ref[0] = (1.0 - ALPHA) * iinv * lax.dot_general(
      x, wsd_ref[...], dn, precision=lax.Precision.HIGHEST,
      preferred_element_type=jnp.float32)
  y_ref[1] = ALPHA * oinv * lax.dot_general(
      x, wds_ref[...], dn, precision=lax.Precision.HIGHEST,
      preferred_element_type=jnp.float32)
  oinv_ref[...] = oinv
  iinv_ref[...] = iinv


def _final_body(acc_ref, oinv_ref, iinv_ref, bsd_ref, bds_ref, out_ref):
  b = (1.0 - ALPHA) * bsd_ref[...] + ALPHA * bds_ref[...]
  out_ref[...] = (oinv_ref[...] * acc_ref[0]
                  + iinv_ref[...] * acc_ref[1]
                  + b[None, :])


def kernel(x, edge_index, W_sd, b_sd, W_ds, b_ds):
  n, d = x.shape
  e = edge_index.shape[1]
  kch = 2 * ((e + NS * CH * 2 - 1) // (NS * CH * 2))
  epw = CH * kch
  e_pad = NS * epw

  row = edge_index[0].astype(jnp.int32)
  col = edge_index[1].astype(jnp.int32)
  pad = jnp.full((e_pad - e,), N_PAD - 1, jnp.int32)
  row_p = jnp.concatenate([row, pad])
  col_p = jnp.concatenate([col, pad])
  dst_idx = jnp.stack([row_p, col_p]).reshape(NC, NS, kch, CH)
  src_idx = jnp.stack([col_p, row_p + N_PAD]).reshape(NC, NS, kch, CH)

  x_pad = jnp.pad(x, ((0, N_PAD - n), (0, 0)))

  deg_call = pl.kernel(
      functools.partial(_deg_body, kch),
      out_type=jax.ShapeDtypeStruct((NC, N_PAD), jnp.float32),
      mesh=_MESH,
      scratch_types=[
          pltpu.VMEM((kch, CH), jnp.int32),
          pltpu.VMEM((CH,), jnp.float32),
          pltpu.VMEM((RPT,), jnp.float32),
          pltpu.VMEM_SHARED((N_PAD,), jnp.float32),
      ],
  )
  deg = deg_call(dst_idx)

  bn = 1024
  grid = (N_PAD // bn,)
  deg3 = deg.reshape(NC, N_PAD, 1)
  y3, oinv, iinv = pl.pallas_call(
      _prep_body,
      grid=grid,
      in_specs=[
          pl.BlockSpec((bn, d), lambda i: (i, 0)),
          pl.BlockSpec((d, d), lambda i: (0, 0)),
          pl.BlockSpec((d, d), lambda i: (0, 0)),
          pl.BlockSpec((bn, 1), lambda i: (i, 0)),
          pl.BlockSpec((bn, 1), lambda i: (i, 0)),
      ],
      out_specs=[
          pl.BlockSpec((NC, bn, d), lambda i: (0, i, 0)),
          pl.BlockSpec((bn, 1), lambda i: (i, 0)),
          pl.BlockSpec((bn, 1), lambda i: (i, 0)),
      ],
      out_shape=[
          jax.ShapeDtypeStruct((NC, N_PAD, d), jnp.float32),
          jax.ShapeDtypeStruct((N_PAD, 1), jnp.float32),
          jax.ShapeDtypeStruct((N_PAD, 1), jnp.float32),
      ],
  )(x_pad, W_sd, W_ds, deg3[0], deg3[1])
  y_cat = y3.reshape(NC * N_PAD, d)

  agg_call = pl.kernel(
      functools.partial(_agg_body, kch, d),
      out_type=jax.ShapeDtypeStruct((NC * N_PAD, d), jnp.float32),
      mesh=_MESH,
      compiler_params=pltpu.CompilerParams(use_tc_tiling_on_sc=False),
      scratch_types=[
          pltpu.VMEM((kch // 2, CH), jnp.int32),
          pltpu.VMEM((kch // 2, CH), jnp.int32),
          pltpu.VMEM((CH, d), jnp.float32),
          pltpu.VMEM((ZR, d), jnp.float32),
          pltpu.VMEM_SHARED((N_PAD, d), jnp.float32),
          pltpu.SemaphoreType.DMA,
      ],
  )
  acc = agg_call(src_idx.reshape(NC, NS, 2, kch // 2, CH),
                 dst_idx.reshape(NC, NS, 2, kch // 2, CH),
                 y_cat).reshape(NC, N_PAD, d)

  out_pad = pl.pallas_call(
      _final_body,
      grid=grid,
      in_specs=[
          pl.BlockSpec((NC, bn, d), lambda i: (0, i, 0)),
          pl.BlockSpec((bn, 1), lambda i: (i, 0)),
          pl.BlockSpec((bn, 1), lambda i: (i, 0)),
          pl.BlockSpec((d,), lambda i: (0,)),
          pl.BlockSpec((d,), lambda i: (0,)),
      ],
      out_specs=pl.BlockSpec((bn, d), lambda i: (i, 0)),
      out_shape=jax.ShapeDtypeStruct((N_PAD, d), jnp.float32),
  )(acc, oinv, iinv, b_sd, b_ds)

  return out_pad[:n]
